# Initial kernel scaffold; baseline (speedup 1.0000x reference)
import jax
import jax.numpy as jnp
from jax.experimental import pallas as pl


def kernel(x, edge_index, emb, W1, att_src1, att_dst1, b1, W2, att_src2, att_dst2, b2):
    raise NotImplementedError("write your pallas kernel here")


# speedup vs baseline: 41.3872x; 41.3872x over previous
import functools

import jax
import jax.numpy as jnp
from jax import lax
from jax.experimental import pallas as pl
from jax.experimental.pallas import tpu as pltpu
from jax.experimental.pallas import tpu_sc as plsc

N_NODES = 100000
N_EDGES = 1600000
IN_DIM = 16
HID = 16
HEADS = 2
OUT_DIM = 32

N_PAD = 102400
ACC_ROWS = 100352
E_TOT = N_EDGES + N_NODES
CHUNK = 256
N_CHUNKS = 416
E_PER_TILE = CHUNK * N_CHUNKS
EP = 16 * E_PER_TILE
SENTINEL = N_NODES

_f32 = jnp.float32
_i32 = jnp.int32


def _emb_gather_body(emb_hbm, nid_hbm, out_hbm, idx_v, rows_v, sem):
    wid = lax.axis_index("s") * 2 + lax.axis_index("c")
    rows_per_w = N_PAD // 32
    idx_rows = rows_per_w // 128

    def body(r, carry):
        pltpu.sync_copy(nid_hbm.at[pl.ds(wid * rows_per_w + r * 128, 128)],
                        idx_v)
        pltpu.async_copy(emb_hbm.at[idx_v],
                         rows_v.at[pl.ds(r * 128, 128)], sem).wait()
        return carry

    lax.fori_loop(0, idx_rows, body, 0)
    pltpu.sync_copy(rows_v, out_hbm.at[pl.ds(wid * rows_per_w, rows_per_w)])


def _emb_gather(emb, nid2d):
    mesh = plsc.VectorSubcoreMesh(core_axis_name="c", subcore_axis_name="s")
    kern = functools.partial(
        pl.kernel,
        mesh=mesh,
        compiler_params=pltpu.CompilerParams(use_tc_tiling_on_sc=False, needs_layout_passes=False),
        out_type=jax.ShapeDtypeStruct((N_PAD, IN_DIM), _f32),
        scratch_types=[
            pltpu.VMEM((128,), _i32),
            pltpu.VMEM((N_PAD // 32, IN_DIM), _f32),
            pltpu.SemaphoreType.DMA,
        ],
    )(_emb_gather_body)
    return kern(emb, nid2d)


def _edge_pass_body(src_hbm, dst_hbm, t0_hbm, t1_hbm, ab0_hbm, ab1_hbm,
                    bias_hbm, out_hbm, sidx, didx, hp, ab_s, ab_d, msg, zbuf,
                    bias_v, iidx, acc, sem):
    c = lax.axis_index("c")
    s = lax.axis_index("s")
    iota = lax.iota(_i32, 16)
    zeros_f = jnp.zeros((16,), _f32)
    rows_per_tile = ACC_ROWS // 16

    pltpu.sync_copy(bias_hbm, bias_v)

    def fill_iidx(base):
        def ib(g, carry):
            e = g * 16 + iota
            plsc.store_scatter(iidx, [e], base + e)
            return carry

        lax.fori_loop(0, 8, ib, 0)

    def zb(g, carry):
        e = g * 16 + iota
        for f in range(17):
            plsc.store_scatter(zbuf, [e, jnp.full((16,), f, _i32)], zeros_f)
        return carry

    lax.fori_loop(0, 8, zb, 0)

    def zacc(k, carry):
        fill_iidx(s * rows_per_tile + k * 128)
        pltpu.sync_copy(zbuf, acc.at[iidx])
        return carry

    lax.fori_loop(0, rows_per_tile // 128, zacc, 0)
    plsc.subcore_barrier()

    def run_half(h, t_hbm, ab_hbm):
        col16 = jnp.full((16,), 16, _i32)
        colA = jnp.full((16,), 0, _i32)
        colB = jnp.full((16,), 1, _i32)

        def chunk_body(j, carry):
            row0 = s * (N_CHUNKS * 2) + j * 2
            pltpu.sync_copy(src_hbm.at[pl.ds(row0, 2)], sidx)
            pltpu.sync_copy(dst_hbm.at[pl.ds(row0, 2)], didx)
            cps = []
            for k in range(2):
                cps.append(pltpu.async_copy(
                    t_hbm.at[sidx.at[k]], hp.at[pl.ds(k * 128, 128)], sem))
                cps.append(pltpu.async_copy(
                    ab_hbm.at[sidx.at[k]], ab_s.at[pl.ds(k * 128, 128)], sem))
                cps.append(pltpu.async_copy(
                    ab_hbm.at[didx.at[k]], ab_d.at[pl.ds(k * 128, 128)], sem))
            for cp in cps:
                cp.wait()

            def gbody(g, carry2):
                e = g * 16 + iota
                a = (plsc.load_gather(ab_s, [e, colA])
                     + plsc.load_gather(ab_d, [e, colB]))
                a = jnp.maximum(a, 0.2 * a)
                ex = jnp.exp(a)
                plsc.store_scatter(msg, [e, col16], ex)
                for f in range(16):
                    colf = jnp.full((16,), f, _i32)
                    hv = plsc.load_gather(hp, [e, colf])
                    plsc.store_scatter(msg, [e, colf], hv * ex)
                return carry2

            lax.fori_loop(0, CHUNK // 16, gbody, 0)
            for k in range(2):
                pltpu.sync_copy(msg.at[pl.ds(k * 128, 128)],
                                acc.at[didx.at[k]], add=True)
            return carry

        lax.fori_loop(0, N_CHUNKS, chunk_body, 0)
        plsc.subcore_barrier()

        bias_bc = [plsc.load_gather(bias_v, [jnp.full((16,), h * 16 + f, _i32)])
                   for f in range(16)]

        def wbody(m, carry):
            r = s * rows_per_tile + m * 128
            fill_iidx(r)
            pltpu.sync_copy(acc.at[iidx], zbuf)

            def wg(g, carry2):
                e = g * 16 + iota
                den = plsc.load_gather(zbuf, [e, col16]) + 1e-16
                inv = 1.0 / den
                for f in range(16):
                    colf = jnp.full((16,), f, _i32)
                    mv = plsc.load_gather(zbuf, [e, colf])
                    plsc.store_scatter(hp, [e, colf], mv * inv + bias_bc[f])
                return carry2

            lax.fori_loop(0, 8, wg, 0)
            pltpu.sync_copy(hp.at[pl.ds(0, 128)],
                            out_hbm.at[h].at[pl.ds(r, 128)])
            return carry

        lax.fori_loop(0, rows_per_tile // 128, wbody, 0)

    @pl.when(c == 0)
    def _():
        run_half(0, t0_hbm, ab0_hbm)

    @pl.when(c == 1)
    def _():
        run_half(1, t1_hbm, ab1_hbm)


def _edge_pass(src2d, dst2d, t0, t1, ab0, ab1, bias):
    mesh = plsc.VectorSubcoreMesh(core_axis_name="c", subcore_axis_name="s")
    kern = functools.partial(
        pl.kernel,
        mesh=mesh,
        compiler_params=pltpu.CompilerParams(use_tc_tiling_on_sc=False, needs_layout_passes=False),
        out_type=jax.ShapeDtypeStruct((2, N_PAD, 16), _f32),
        scratch_types=[
            pltpu.VMEM((2, 128), _i32),
            pltpu.VMEM((2, 128), _i32),
            pltpu.VMEM((CHUNK, 16), _f32),
            pltpu.VMEM((CHUNK, 2), _f32),
            pltpu.VMEM((CHUNK, 2), _f32),
            pltpu.VMEM((CHUNK, 17), _f32),
            pltpu.VMEM((128, 17), _f32),
            pltpu.VMEM((32,), _f32),
            pltpu.VMEM((128,), _i32),
            pltpu.VMEM_SHARED((ACC_ROWS, 17), _f32),
            pltpu.SemaphoreType.DMA,
        ],
    )(_edge_pass_body)
    return kern(src2d, dst2d, t0, t1, ab0, ab1, bias)


def _proj1_body(h_ref, w_ref, o_ref):
    o_ref[...] = jnp.dot(h_ref[...], w_ref[...], preferred_element_type=_f32)


def _proj1(h, w_ext):
    return pl.pallas_call(
        _proj1_body,
        grid=(N_PAD // 512,),
        in_specs=[
            pl.BlockSpec((512, IN_DIM), lambda i: (i, 0)),
            pl.BlockSpec((IN_DIM, 36), lambda i: (0, 0)),
        ],
        out_specs=pl.BlockSpec((512, 36), lambda i: (i, 0)),
        out_shape=jax.ShapeDtypeStruct((N_PAD, 36), _f32),
    )(h, w_ext)


def _proj2_body(a_ref, b_ref, w_ref, o_ref):
    y = jnp.concatenate([a_ref[0], a_ref[1]], axis=-1)
    y = jnp.maximum(y + b_ref[...], 0.0)
    o_ref[...] = jnp.dot(y, w_ref[...], preferred_element_type=_f32)


def _proj2(y_halves, b1, w_ext2):
    return pl.pallas_call(
        _proj2_body,
        grid=(N_PAD // 512,),
        in_specs=[
            pl.BlockSpec((2, 512, 16), lambda i: (0, i, 0)),
            pl.BlockSpec((1, 32), lambda i: (0, 0)),
            pl.BlockSpec((32, 34), lambda i: (0, 0)),
        ],
        out_specs=pl.BlockSpec((512, 34), lambda i: (i, 0)),
        out_shape=jax.ShapeDtypeStruct((N_PAD, 34), _f32),
    )(y_halves, b1, w_ext2)


def kernel(x, edge_index, emb, W1, att_src1, att_dst1, b1,
           W2, att_src2, att_dst2, b2):
    node_ids = x[:, 0]
    nid_pad = jnp.concatenate(
        [node_ids, jnp.zeros((N_PAD - N_NODES,), _i32)])

    loop = jnp.arange(N_NODES, dtype=_i32)
    pad = jnp.full((EP - E_TOT,), SENTINEL, _i32)
    src2d = jnp.concatenate([edge_index[0], loop, pad]).reshape(-1, 128)
    dst2d = jnp.concatenate([edge_index[1], loop, pad]).reshape(-1, 128)

    w1h = W1.reshape(IN_DIM, HEADS, HID)
    a_s1 = jnp.einsum("khc,hc->kh", w1h, att_src1[0])
    a_d1 = jnp.einsum("khc,hc->kh", w1h, att_dst1[0])
    w_ext1 = jnp.concatenate(
        [W1, a_s1[:, 0:1], a_d1[:, 0:1], a_s1[:, 1:2], a_d1[:, 1:2]], axis=1)

    a_s2 = W2 @ att_src2[0, 0]
    a_d2 = W2 @ att_dst2[0, 0]
    w_ext2 = jnp.concatenate([W2, a_s2[:, None], a_d2[:, None]], axis=1)

    h = _emb_gather(emb, nid_pad)
    hp1 = _proj1(h, w_ext1)
    y1 = _edge_pass(src2d, dst2d,
                    hp1[:, 0:16], hp1[:, 16:32],
                    hp1[:, 32:34], hp1[:, 34:36],
                    jnp.zeros((32,), _f32))

    hp2 = _proj2(y1, b1.reshape(1, 32), w_ext2)
    y2 = _edge_pass(src2d, dst2d,
                    hp2[:, 0:16], hp2[:, 16:32],
                    hp2[:, 32:34], hp2[:, 32:34],
                    b2)

    return jnp.concatenate([y2[0, :N_NODES], y2[1, :N_NODES]], axis=1)


# speedup vs baseline: 58.2871x; 1.4083x over previous
import functools

import jax
import jax.numpy as jnp
from jax import lax
from jax.experimental import pallas as pl
from jax.experimental.pallas import tpu as pltpu
from jax.experimental.pallas import tpu_sc as plsc

N_NODES = 100000
N_EDGES = 1600000
IN_DIM = 16
HID = 16

N_PAD = 102400
ACC_ROWS = 100352
E_TOT = N_EDGES + N_NODES
CHUNK = 128
SUPER = 8
N_SUPER = 104
E_PER_TILE = CHUNK * SUPER * N_SUPER
EP = 16 * E_PER_TILE
SENTINEL = N_NODES

_f32 = jnp.float32
_i32 = jnp.int32


def _emb_gather_body(emb_hbm, nid_hbm, out_hbm, idx_v, rows_v, sem):
    wid = lax.axis_index("s") * 2 + lax.axis_index("c")
    rows_per_w = N_PAD // 32
    idx_rows = rows_per_w // 128

    def body(r, carry):
        pltpu.sync_copy(nid_hbm.at[pl.ds(wid * rows_per_w + r * 128, 128)],
                        idx_v)
        pltpu.async_copy(emb_hbm.at[idx_v],
                         rows_v.at[pl.ds(r * 128, 128)], sem).wait()
        return carry

    lax.fori_loop(0, idx_rows, body, 0)
    pltpu.sync_copy(rows_v, out_hbm.at[pl.ds(wid * rows_per_w, rows_per_w)])


def _emb_gather(emb, nid1d):
    mesh = plsc.VectorSubcoreMesh(core_axis_name="c", subcore_axis_name="s")
    kern = functools.partial(
        pl.kernel,
        mesh=mesh,
        compiler_params=pltpu.CompilerParams(
            use_tc_tiling_on_sc=False, needs_layout_passes=False),
        out_type=jax.ShapeDtypeStruct((N_PAD, IN_DIM), _f32),
        scratch_types=[
            pltpu.VMEM((128,), _i32),
            pltpu.VMEM((N_PAD // 32, IN_DIM), _f32),
            pltpu.SemaphoreType.DMA,
        ],
    )(_emb_gather_body)
    return kern(emb, nid1d)


def _edge_pass_body(src_hbm, dst_hbm, t0_hbm, t1_hbm, ab0_hbm, ab1_hbm,
                    bias_hbm, out_hbm, sidx, didx, hp0, hp1, abd0, abd1,
                    msg0, msg1, bias_v, iidx, acc,
                    gsemA, gsemB, ssemA, ssemB):
    c = lax.axis_index("c")
    s = lax.axis_index("s")
    iota = lax.iota(_i32, 16)
    zeros_f = jnp.zeros((16,), _f32)
    rows_per_tile = ACC_ROWS // 16
    col16 = jnp.full((16,), 16, _i32)
    col1 = jnp.full((16,), 1, _i32)

    pltpu.sync_copy(bias_hbm, bias_v)

    def fill_iidx(base):
        def ib(g, carry):
            e = g * 16 + iota
            plsc.store_scatter(iidx, [e], base + e)
            return carry

        lax.fori_loop(0, 8, ib, 0)

    def zb(g, carry):
        e = g * 16 + iota
        for f in range(17):
            plsc.store_scatter(msg0, [e, jnp.full((16,), f, _i32)], zeros_f)
        return carry

    lax.fori_loop(0, 8, zb, 0)

    def zacc(k, carry):
        fill_iidx(s * rows_per_tile + k * 128)
        pltpu.sync_copy(msg0, acc.at[iidx])
        return carry

    lax.fori_loop(0, rows_per_tile // 128, zacc, 0)
    plsc.subcore_barrier()

    def run_half(h, t_hbm, ab_hbm):
        hps = [hp0, hp1]
        abds = [abd0, abd1]
        msgs = [msg0, msg1]
        gsems = [gsemA, gsemB]
        ssems = [ssemA, ssemB]

        def compute_chunk(p):
            def gbody(g, carry2):
                e = g * 16 + iota
                a = (plsc.load_gather(hps[p], [e, col16])
                     + plsc.load_gather(abds[p], [e, col1]))
                a = jnp.maximum(a, 0.2 * a)
                ex = jnp.exp(a)
                plsc.store_scatter(msgs[p], [e, col16], ex)
                for f in range(16):
                    colf = jnp.full((16,), f, _i32)
                    hv = plsc.load_gather(hps[p], [e, colf])
                    plsc.store_scatter(msgs[p], [e, colf], hv * ex)
                return carry2

            lax.fori_loop(0, CHUNK // 16, gbody, 0)

        def super_body(j, carry):
            row0 = s * (N_SUPER * SUPER) + j * SUPER
            pltpu.sync_copy(src_hbm.at[pl.ds(row0, SUPER)], sidx)
            pltpu.sync_copy(dst_hbm.at[pl.ds(row0, SUPER)], didx)
            gcps = {}
            scps = {}
            for k in range(SUPER):
                p = k % 2
                gcps[k] = [
                    pltpu.async_copy(t_hbm.at[sidx.at[k]], hps[p], gsems[p]),
                    pltpu.async_copy(ab_hbm.at[didx.at[k]], abds[p], gsems[p]),
                ]
                if k > 0:
                    q = (k - 1) % 2
                    for cp in gcps.pop(k - 1):
                        cp.wait()
                    if k > 1:
                        scps.pop(k - 2).wait()
                    compute_chunk(q)
                    scps[k - 1] = pltpu.async_copy(
                        msgs[q], acc.at[didx.at[k - 1]], ssems[q])
            q = (SUPER - 1) % 2
            for cp in gcps.pop(SUPER - 1):
                cp.wait()
            scps.pop(SUPER - 2).wait()
            compute_chunk(q)
            pltpu.async_copy(msgs[q], acc.at[didx.at[SUPER - 1]],
                             ssems[q]).wait()
            return carry

        lax.fori_loop(0, N_SUPER, super_body, 0)
        plsc.subcore_barrier()

        bias_bc = [plsc.load_gather(bias_v, [jnp.full((16,), h * 16 + f, _i32)])
                   for f in range(16)]

        def wbody(m, carry):
            r = s * rows_per_tile + m * 128
            fill_iidx(r)
            pltpu.sync_copy(acc.at[iidx], msg0)

            def wg(g, carry2):
                e = g * 16 + iota
                den = plsc.load_gather(msg0, [e, col16]) + 1e-16
                inv = 1.0 / den
                for f in range(16):
                    colf = jnp.full((16,), f, _i32)
                    mv = plsc.load_gather(msg0, [e, colf])
                    plsc.store_scatter(hp0, [e, colf], mv * inv + bias_bc[f])
                return carry2

            lax.fori_loop(0, 8, wg, 0)
            pltpu.sync_copy(hp0, out_hbm.at[h].at[pl.ds(r, 128)])
            return carry

        lax.fori_loop(0, rows_per_tile // 128, wbody, 0)

    @pl.when(c == 0)
    def _():
        run_half(0, t0_hbm, ab0_hbm)

    @pl.when(c == 1)
    def _():
        run_half(1, t1_hbm, ab1_hbm)


def _edge_pass(src2d, dst2d, t0, t1, ab0, ab1, bias):
    mesh = plsc.VectorSubcoreMesh(core_axis_name="c", subcore_axis_name="s")
    kern = functools.partial(
        pl.kernel,
        mesh=mesh,
        compiler_params=pltpu.CompilerParams(
            use_tc_tiling_on_sc=False, needs_layout_passes=False),
        out_type=jax.ShapeDtypeStruct((2, N_PAD, 18), _f32),
        scratch_types=[
            pltpu.VMEM((SUPER, 128), _i32),
            pltpu.VMEM((SUPER, 128), _i32),
            pltpu.VMEM((CHUNK, 18), _f32),
            pltpu.VMEM((CHUNK, 18), _f32),
            pltpu.VMEM((CHUNK, 2), _f32),
            pltpu.VMEM((CHUNK, 2), _f32),
            pltpu.VMEM((CHUNK, 17), _f32),
            pltpu.VMEM((CHUNK, 17), _f32),
            pltpu.VMEM((32,), _f32),
            pltpu.VMEM((128,), _i32),
            pltpu.VMEM_SHARED((ACC_ROWS, 17), _f32),
            pltpu.SemaphoreType.DMA,
            pltpu.SemaphoreType.DMA,
            pltpu.SemaphoreType.DMA,
            pltpu.SemaphoreType.DMA,
        ],
    )(_edge_pass_body)
    return kern(src2d, dst2d, t0, t1, ab0, ab1, bias)


def _proj1_body(h_ref, w_ref, o_ref):
    o_ref[...] = jnp.dot(h_ref[...], w_ref[...], preferred_element_type=_f32)


def _proj1(h, w_ext):
    return pl.pallas_call(
        _proj1_body,
        grid=(N_PAD // 512,),
        in_specs=[
            pl.BlockSpec((512, IN_DIM), lambda i: (i, 0)),
            pl.BlockSpec((IN_DIM, 36), lambda i: (0, 0)),
        ],
        out_specs=pl.BlockSpec((512, 36), lambda i: (i, 0)),
        out_shape=jax.ShapeDtypeStruct((N_PAD, 36), _f32),
    )(h, w_ext)


def _proj2_body(a_ref, b_ref, w_ref, o_ref):
    y = jnp.concatenate([a_ref[0, :, 0:16], a_ref[1, :, 0:16]], axis=-1)
    y = jnp.maximum(y + b_ref[...], 0.0)
    o_ref[...] = jnp.dot(y, w_ref[...], preferred_element_type=_f32)


def _proj2(y_halves, b1, w_ext2):
    return pl.pallas_call(
        _proj2_body,
        grid=(N_PAD // 512,),
        in_specs=[
            pl.BlockSpec((2, 512, 18), lambda i: (0, i, 0)),
            pl.BlockSpec((1, 32), lambda i: (0, 0)),
            pl.BlockSpec((32, 36), lambda i: (0, 0)),
        ],
        out_specs=pl.BlockSpec((512, 36), lambda i: (i, 0)),
        out_shape=jax.ShapeDtypeStruct((N_PAD, 36), _f32),
    )(y_halves, b1, w_ext2)


def kernel(x, edge_index, emb, W1, att_src1, att_dst1, b1,
           W2, att_src2, att_dst2, b2):
    node_ids = x[:, 0]
    nid_pad = jnp.concatenate(
        [node_ids, jnp.zeros((N_PAD - N_NODES,), _i32)])

    loop = jnp.arange(N_NODES, dtype=_i32)
    pad = jnp.full((EP - E_TOT,), SENTINEL, _i32)
    src2d = jnp.concatenate([edge_index[0], loop, pad]).reshape(-1, 128)
    dst2d = jnp.concatenate([edge_index[1], loop, pad]).reshape(-1, 128)

    w1h = W1.reshape(IN_DIM, 2, HID)
    a_s1 = jnp.einsum("khc,hc->kh", w1h, att_src1[0])
    a_d1 = jnp.einsum("khc,hc->kh", w1h, att_dst1[0])
    w_ext1 = jnp.concatenate(
        [W1[:, 0:16], a_s1[:, 0:1], a_d1[:, 0:1],
         W1[:, 16:32], a_s1[:, 1:2], a_d1[:, 1:2]], axis=1)

    a_s2 = (W2 @ att_src2[0, 0])[:, None]
    a_d2 = (W2 @ att_dst2[0, 0])[:, None]
    w_ext2 = jnp.concatenate(
        [W2[:, 0:16], a_s2, a_d2, W2[:, 16:32], a_s2, a_d2], axis=1)

    h = _emb_gather(emb, nid_pad)
    hp1 = _proj1(h, w_ext1)
    y1 = _edge_pass(src2d, dst2d,
                    hp1[:, 0:18], hp1[:, 18:36],
                    hp1[:, 16:18], hp1[:, 34:36],
                    jnp.zeros((32,), _f32))

    hp2 = _proj2(y1, b1.reshape(1, 32), w_ext2)
    y2 = _edge_pass(src2d, dst2d,
                    hp2[:, 0:18], hp2[:, 18:36],
                    hp2[:, 16:18], hp2[:, 34:36],
                    b2)

    return jnp.concatenate([y2[0, :N_NODES, 0:16], y2[1, :N_NODES, 0:16]],
                           axis=1)


# speedup vs baseline: 60.4144x; 1.0365x over previous
import functools

import jax
import jax.numpy as jnp
from jax import lax
from jax.experimental import pallas as pl
from jax.experimental.pallas import tpu as pltpu
from jax.experimental.pallas import tpu_sc as plsc

N_NODES = 100000
N_EDGES = 1600000
IN_DIM = 16
HID = 16

N_PAD = 102400
ACC_ROWS = 100352
E_TOT = N_EDGES + N_NODES
CHUNK = 128
SUPER = 8
N_SUPER = 104
E_PER_TILE = CHUNK * SUPER * N_SUPER
EP = 16 * E_PER_TILE
SENTINEL = N_NODES

_f32 = jnp.float32
_i32 = jnp.int32


def _emb_gather_body(emb_hbm, nid_hbm, out_hbm, idx_v, rows_v, sem):
    wid = lax.axis_index("s") * 2 + lax.axis_index("c")
    rows_per_w = N_PAD // 32
    idx_rows = rows_per_w // 128

    def body(r, carry):
        pltpu.sync_copy(nid_hbm.at[pl.ds(wid * rows_per_w + r * 128, 128)],
                        idx_v)
        pltpu.async_copy(emb_hbm.at[idx_v],
                         rows_v.at[pl.ds(r * 128, 128)], sem).wait()
        return carry

    lax.fori_loop(0, idx_rows, body, 0)
    pltpu.sync_copy(rows_v, out_hbm.at[pl.ds(wid * rows_per_w, rows_per_w)])


def _emb_gather(emb, nid1d):
    mesh = plsc.VectorSubcoreMesh(core_axis_name="c", subcore_axis_name="s")
    kern = functools.partial(
        pl.kernel,
        mesh=mesh,
        compiler_params=pltpu.CompilerParams(
            use_tc_tiling_on_sc=False, needs_layout_passes=False),
        out_type=jax.ShapeDtypeStruct((N_PAD, IN_DIM), _f32),
        scratch_types=[
            pltpu.VMEM((128,), _i32),
            pltpu.VMEM((N_PAD // 32, IN_DIM), _f32),
            pltpu.SemaphoreType.DMA,
        ],
    )(_emb_gather_body)
    return kern(emb, nid1d)


def _edge_pass_body(src_hbm, dst_hbm, t0_hbm, t1_hbm, ab0_hbm, ab1_hbm,
                    bias_hbm, out_hbm, sidx, didx, hp0, hp1, abd0, abd1,
                    msg0, msg1, bias_v, iidx, acc,
                    gsemA, gsemB, ssemA, ssemB):
    c = lax.axis_index("c")
    s = lax.axis_index("s")
    iota = lax.iota(_i32, 16)
    zeros_f = jnp.zeros((16,), _f32)
    rows_per_tile = ACC_ROWS // 16
    col16 = jnp.full((16,), 16, _i32)
    col1 = jnp.full((16,), 1, _i32)

    pltpu.sync_copy(bias_hbm, bias_v)

    def fill_iidx(base):
        def ib(g, carry):
            e = g * 16 + iota
            plsc.store_scatter(iidx, [e], base + e)
            return carry

        lax.fori_loop(0, 8, ib, 0)

    def zb(g, carry):
        e = g * 16 + iota
        for f in range(17):
            plsc.store_scatter(msg0, [e, jnp.full((16,), f, _i32)], zeros_f)
        return carry

    lax.fori_loop(0, 8, zb, 0)

    def zacc(k, carry):
        fill_iidx(s * rows_per_tile + k * 128)
        pltpu.sync_copy(msg0, acc.at[iidx])
        return carry

    lax.fori_loop(0, rows_per_tile // 128, zacc, 0)
    plsc.subcore_barrier()

    def run_half(h, t_hbm, ab_hbm):
        hps = [hp0, hp1]
        abds = [abd0, abd1]
        msgs = [msg0, msg1]
        gsems = [gsemA, gsemB]
        ssems = [ssemA, ssemB]

        def compute_chunk(p):
            def gbody(g, carry2):
                e = g * 16 + iota
                a = (plsc.load_gather(hps[p], [e, col16])
                     + plsc.load_gather(abds[p], [e, col1]))
                a = jnp.maximum(a, 0.2 * a)
                ex = jnp.exp(a)
                plsc.store_scatter(msgs[p], [e, col16], ex)
                for f in range(16):
                    colf = jnp.full((16,), f, _i32)
                    hv = plsc.load_gather(hps[p], [e, colf])
                    plsc.store_scatter(msgs[p], [e, colf], hv * ex)
                return carry2

            lax.fori_loop(0, CHUNK // 16, gbody, 0)

        def super_body(j, carry):
            row0 = s * (N_SUPER * SUPER) + j * SUPER
            pltpu.sync_copy(src_hbm.at[pl.ds(row0, SUPER)], sidx)
            pltpu.sync_copy(dst_hbm.at[pl.ds(row0, SUPER)], didx)
            gcps = {}
            scps = {}
            for k in range(SUPER):
                p = k % 2
                gcps[k] = [
                    pltpu.async_copy(t_hbm.at[sidx.at[k]], hps[p], gsems[p]),
                    pltpu.async_copy(ab_hbm.at[didx.at[k]], abds[p], gsems[p]),
                ]
                if k > 0:
                    q = (k - 1) % 2
                    for cp in gcps.pop(k - 1):
                        cp.wait()
                    if k > 1:
                        scps.pop(k - 2).wait()
                    compute_chunk(q)
                    scps[k - 1] = pltpu.async_copy(
                        msgs[q], acc.at[didx.at[k - 1]], ssems[q])
            q = (SUPER - 1) % 2
            for cp in gcps.pop(SUPER - 1):
                cp.wait()
            scps.pop(SUPER - 2).wait()
            compute_chunk(q)
            pltpu.async_copy(msgs[q], acc.at[didx.at[SUPER - 1]],
                             ssems[q]).wait()
            return carry

        lax.fori_loop(0, N_SUPER, super_body, 0)
        plsc.subcore_barrier()

        bias_bc = [plsc.load_gather(bias_v, [jnp.full((16,), h * 16 + f, _i32)])
                   for f in range(16)]

        def wbody(m, carry):
            r = s * rows_per_tile + m * 128
            fill_iidx(r)
            pltpu.sync_copy(acc.at[iidx], msg0)

            def wg(g, carry2):
                e = g * 16 + iota
                den = plsc.load_gather(msg0, [e, col16]) + 1e-16
                inv = 1.0 / den
                for f in range(16):
                    colf = jnp.full((16,), f, _i32)
                    mv = plsc.load_gather(msg0, [e, colf])
                    plsc.store_scatter(hp0, [e, colf], mv * inv + bias_bc[f])
                return carry2

            lax.fori_loop(0, 8, wg, 0)
            pltpu.sync_copy(hp0, out_hbm.at[h].at[pl.ds(r, 128)])
            return carry

        lax.fori_loop(0, rows_per_tile // 128, wbody, 0)

    @pl.when(c == 0)
    def _():
        run_half(0, t0_hbm, ab0_hbm)

    @pl.when(c == 1)
    def _():
        run_half(1, t1_hbm, ab1_hbm)


def _edge_pass(src2d, dst2d, t0, t1, ab0, ab1, bias):
    mesh = plsc.VectorSubcoreMesh(core_axis_name="c", subcore_axis_name="s")
    kern = functools.partial(
        pl.kernel,
        mesh=mesh,
        compiler_params=pltpu.CompilerParams(
            use_tc_tiling_on_sc=False, needs_layout_passes=False),
        out_type=jax.ShapeDtypeStruct((2, N_PAD, 18), _f32),
        scratch_types=[
            pltpu.VMEM((SUPER, 128), _i32),
            pltpu.VMEM((SUPER, 128), _i32),
            pltpu.VMEM((CHUNK, 18), _f32),
            pltpu.VMEM((CHUNK, 18), _f32),
            pltpu.VMEM((CHUNK, 2), _f32),
            pltpu.VMEM((CHUNK, 2), _f32),
            pltpu.VMEM((CHUNK, 17), _f32),
            pltpu.VMEM((CHUNK, 17), _f32),
            pltpu.VMEM((32,), _f32),
            pltpu.VMEM((128,), _i32),
            pltpu.VMEM_SHARED((ACC_ROWS, 17), _f32),
            pltpu.SemaphoreType.DMA,
            pltpu.SemaphoreType.DMA,
            pltpu.SemaphoreType.DMA,
            pltpu.SemaphoreType.DMA,
        ],
    )(_edge_pass_body)
    return kern(src2d, dst2d, t0, t1, ab0, ab1, bias)


_PROJ_OUT_SPECS = [
    pl.BlockSpec((512, 18), lambda i: (i, 0)),
    pl.BlockSpec((512, 18), lambda i: (i, 0)),
    pl.BlockSpec((512, 2), lambda i: (i, 0)),
    pl.BlockSpec((512, 2), lambda i: (i, 0)),
]
_PROJ_OUT_SHAPE = [
    jax.ShapeDtypeStruct((N_PAD, 18), _f32),
    jax.ShapeDtypeStruct((N_PAD, 18), _f32),
    jax.ShapeDtypeStruct((N_PAD, 2), _f32),
    jax.ShapeDtypeStruct((N_PAD, 2), _f32),
]


def _split_out(hp, t0_ref, t1_ref, ab0_ref, ab1_ref):
    t0_ref[...] = hp[:, 0:18]
    t1_ref[...] = hp[:, 18:36]
    ab0_ref[...] = hp[:, 16:18]
    ab1_ref[...] = hp[:, 34:36]


def _proj1_body(h_ref, w_ref, t0_ref, t1_ref, ab0_ref, ab1_ref):
    hp = jnp.dot(h_ref[...], w_ref[...], preferred_element_type=_f32)
    _split_out(hp, t0_ref, t1_ref, ab0_ref, ab1_ref)


def _proj1(h, w_ext):
    return pl.pallas_call(
        _proj1_body,
        grid=(N_PAD // 512,),
        in_specs=[
            pl.BlockSpec((512, IN_DIM), lambda i: (i, 0)),
            pl.BlockSpec((IN_DIM, 36), lambda i: (0, 0)),
        ],
        out_specs=_PROJ_OUT_SPECS,
        out_shape=_PROJ_OUT_SHAPE,
    )(h, w_ext)


def _proj2_body(a_ref, b_ref, w_ref, t0_ref, t1_ref, ab0_ref, ab1_ref):
    y = jnp.concatenate([a_ref[0, :, 0:16], a_ref[1, :, 0:16]], axis=-1)
    y = jnp.maximum(y + b_ref[...], 0.0)
    hp = jnp.dot(y, w_ref[...], preferred_element_type=_f32)
    _split_out(hp, t0_ref, t1_ref, ab0_ref, ab1_ref)


def _proj2(y_halves, b1, w_ext2):
    return pl.pallas_call(
        _proj2_body,
        grid=(N_PAD // 512,),
        in_specs=[
            pl.BlockSpec((2, 512, 18), lambda i: (0, i, 0)),
            pl.BlockSpec((1, 32), lambda i: (0, 0)),
            pl.BlockSpec((32, 36), lambda i: (0, 0)),
        ],
        out_specs=_PROJ_OUT_SPECS,
        out_shape=_PROJ_OUT_SHAPE,
    )(y_halves, b1, w_ext2)


def kernel(x, edge_index, emb, W1, att_src1, att_dst1, b1,
           W2, att_src2, att_dst2, b2):
    node_ids = x[:, 0]
    nid_pad = jnp.concatenate(
        [node_ids, jnp.zeros((N_PAD - N_NODES,), _i32)])

    loop = jnp.arange(N_NODES, dtype=_i32)
    pad = jnp.full((EP - E_TOT,), SENTINEL, _i32)
    src2d = jnp.concatenate([edge_index[0], loop, pad]).reshape(-1, 128)
    dst2d = jnp.concatenate([edge_index[1], loop, pad]).reshape(-1, 128)

    w1h = W1.reshape(IN_DIM, 2, HID)
    a_s1 = jnp.einsum("khc,hc->kh", w1h, att_src1[0])
    a_d1 = jnp.einsum("khc,hc->kh", w1h, att_dst1[0])
    w_ext1 = jnp.concatenate(
        [W1[:, 0:16], a_s1[:, 0:1], a_d1[:, 0:1],
         W1[:, 16:32], a_s1[:, 1:2], a_d1[:, 1:2]], axis=1)

    a_s2 = (W2 @ att_src2[0, 0])[:, None]
    a_d2 = (W2 @ att_dst2[0, 0])[:, None]
    w_ext2 = jnp.concatenate(
        [W2[:, 0:16], a_s2, a_d2, W2[:, 16:32], a_s2, a_d2], axis=1)

    h = _emb_gather(emb, nid_pad)
    t0a, t1a, ab0a, ab1a = _proj1(h, w_ext1)
    y1 = _edge_pass(src2d, dst2d, t0a, t1a, ab0a, ab1a,
                    jnp.zeros((32,), _f32))

    t0b, t1b, ab0b, ab1b = _proj2(y1, b1.reshape(1, 32), w_ext2)
    y2 = _edge_pass(src2d, dst2d, t0b, t1b, ab0b, ab1b,
                    b2)

    return jnp.concatenate([y2[0, :N_NODES, 0:16], y2[1, :N_NODES, 0:16]],
                           axis=1)
